# speedup vs baseline: 9.1183x; 9.1183x over previous
import functools
import math

import jax
import jax.numpy as jnp
from jax import lax
from jax.experimental import pallas as pl
from jax.experimental.pallas import tpu as pltpu
from jax.experimental.pallas import tpu_sc as plsc

D_MODEL = 128
SCALE = math.sqrt(float(D_MODEL))
NC, NS = 2, 16
NW = NC * NS
CHUNK = 128
NBUF = 4
LANES = 16


@functools.lru_cache(maxsize=None)
def _build_sc_gather(bs: int):
    nchunks_total = bs // CHUNK
    nch = nchunks_total // NW
    mesh = plsc.VectorSubcoreMesh(core_axis_name="c", subcore_axis_name="s")

    def body(idx_hbm, table_hbm, out_hbm, idx_v, *rest):
        bufs = rest[:NBUF]
        gsems = rest[NBUF:2 * NBUF]
        ssems = rest[2 * NBUF:3 * NBUF]

        wid = lax.axis_index("s") * NC + lax.axis_index("c")
        chunk0 = wid * nch
        row0 = chunk0 * CHUNK

        pltpu.sync_copy(idx_hbm.at[pl.ds(chunk0, nch)], idx_v)

        def start_gather(b, c):
            pltpu.async_copy(table_hbm.at[idx_v.at[c]], bufs[b], gsems[b])

        def wait_gather(b):
            pltpu.make_async_copy(
                table_hbm.at[idx_v.at[0]], bufs[b], gsems[b]).wait()

        def start_store(b, c):
            pltpu.async_copy(
                bufs[b], out_hbm.at[pl.ds(row0 + c * CHUNK, CHUNK)], ssems[b])

        def wait_store(b):
            pltpu.make_async_copy(
                bufs[b], out_hbm.at[pl.ds(row0, CHUNK)], ssems[b]).wait()

        for b in range(NBUF):
            start_gather(b, b)

        def ring(g, carry):
            for b in range(NBUF):
                c = g * NBUF + b
                wait_gather(b)

                def scale_row(i, carry2, b=b):
                    for j in range(D_MODEL // LANES):
                        sl = pl.ds(j * LANES, LANES)
                        bufs[b][i, sl] = bufs[b][i, sl] * SCALE
                    return carry2

                lax.fori_loop(0, CHUNK, scale_row, 0, unroll=False)
                start_store(b, c)

                @pl.when(c + NBUF < nch)
                def _(b=b, c=c):
                    wait_store(b)
                    start_gather(b, c + NBUF)
            return carry

        lax.fori_loop(0, nch // NBUF, ring, 0, unroll=False)
        for b in range(NBUF):
            wait_store(b)

    return pl.kernel(
        body,
        out_type=jax.ShapeDtypeStruct((bs, D_MODEL), jnp.float32),
        mesh=mesh,
        scratch_types=(
            [pltpu.VMEM((nch, CHUNK), jnp.int32)]
            + [pltpu.VMEM((CHUNK, D_MODEL), jnp.float32) for _ in range(NBUF)]
            + [pltpu.SemaphoreType.DMA for _ in range(2 * NBUF)]
        ),
    )


def kernel(x, table):
    b, s = x.shape
    bs = b * s
    idx2 = x.astype(jnp.int32).reshape(bs // CHUNK, CHUNK)
    out = _build_sc_gather(bs)(idx2, table)
    return out.reshape(b, s, D_MODEL)
